# baseline (speedup 1.0000x reference)
import functools

import jax
import jax.numpy as jnp
import numpy as np
from jax import lax
from jax.experimental import pallas as pl
from jax.experimental.pallas import tpu as pltpu
from jax.experimental.pallas import tpu_sc as plsc

_N = 1024
_B = 512
_L = 16

_OMU = float(np.float32(1.0) - np.array(1064475214, np.uint32).view(np.float32))


def _detect_body(posT, pos, rr, mr, rmc, rec_ref, lt_ref):
    g = pl.program_id(0)
    i0 = g * _B
    jj = lax.broadcasted_iota(jnp.int32, (_B, _N), 1)
    ii = i0 + lax.broadcasted_iota(jnp.int32, (_B, _N), 0)

    @pl.when(g == 0)
    def _():
        rows = lax.broadcasted_iota(jnp.int32, (_N, _N), 0)
        cols = lax.broadcasted_iota(jnp.int32, (_N, _N), 1)
        lt_ref[...] = (rows <= cols).astype(jnp.bfloat16)

    px_row = posT[0:1, :]
    py_row = posT[1:2, :]
    px_col = pos[:, 0:1]
    py_col = pos[:, 1:2]

    dx = px_row - px_col
    dy = py_row - py_col
    dd = (dx * dx + dy * dy) + 1e-12
    dist = jnp.sqrt(dd)
    rc = rmc[:, 0:1]
    mc = rmc[:, 1:2]
    pen = (rc + rr[...]) - dist
    mask = (pen > 0.0) & (jj < ii)

    mif = mask.astype(jnp.float32)
    k = lax.dot_general(
        mask.astype(jnp.bfloat16), lt_ref[...],
        (((1,), (0,)), ((), ())), preferred_element_type=jnp.float32)

    cnt = k[:, _N - 1:_N]
    cnt_f = jnp.maximum(cnt, 1.0)
    q = 1.0 / cnt_f
    c = k * q
    r = (cnt_f * q) * _OMU
    onehot = (c >= r) & (((k - mif) * q) < r)
    ohf = onehot.astype(jnp.float32)

    jsel = jnp.sum(onehot.astype(jnp.int32) * jj, axis=1, keepdims=True)
    dxs = jnp.sum(ohf * dx, axis=1, keepdims=True)
    dys = jnp.sum(ohf * dy, axis=1, keepdims=True)
    dists = jnp.sum(ohf * dist, axis=1, keepdims=True)
    pens = jnp.sum(ohf * pen, axis=1, keepdims=True)
    imr = 1.0 / mr[...]
    imj = jnp.sum(ohf * imr, axis=1, keepdims=True)
    imi = 1.0 / mc

    dists = jnp.where(dists > 0.0, dists, 1.0)
    pvx = dxs / dists * pens
    pvy = dys / dists * pens
    nden = jnp.sqrt(pvx * pvx + pvy * pvy) + 1e-12
    nxv = pvx / nden
    nyv = pvy / nden
    s = imi + imj
    avi = 1.5 * imi / s
    avj = 1.5 * imj / s
    corrx = (0.8 * pvx) / s
    corry = (0.8 * pvy) / s

    z = jnp.zeros((_B, 1), jnp.float32)
    rec = jnp.concatenate(
        [-(corrx * imi), -(corry * imi), avi * nxv, avi * nyv,
         corrx * imj, corry * imj, -(avj * nxv), -(avj * nyv),
         -nxv, -nyv, nxv, nyv,
         jsel.astype(jnp.float32), (cnt > 0.0).astype(jnp.float32),
         z, z], axis=1)
    rec_ref[...] = rec


def _detect(posT, pos, radii, masses):
    row2 = pl.BlockSpec((2, _N), lambda g: (0, 0))
    row1 = pl.BlockSpec((1, _N), lambda g: (0, 0))
    col2 = pl.BlockSpec((_B, 2), lambda g: (g, 0))
    col1 = pl.BlockSpec((_B, 1), lambda g: (g, 0))
    outw = pl.BlockSpec((_B, _L), lambda g: (g, 0))
    fw = jax.ShapeDtypeStruct((_N, _L), jnp.float32)
    rm = jnp.concatenate([radii[:, None], masses[:, None]], axis=1)
    return pl.pallas_call(
        _detect_body,
        grid=(_N // _B,),
        in_specs=[row2, col2, row1, row1, col2],
        out_specs=[outw],
        out_shape=[fw],
        scratch_shapes=[pltpu.VMEM((_N, _N), jnp.bfloat16)],
    )(posT, pos, radii.reshape(1, _N), masses.reshape(1, _N), rm)[0]


def _resolve_body(pos_hbm, vel_hbm, rec_hbm,
                  pos_out, vel_out,
                  s_v, rec_v, civ_v):
    cid = lax.axis_index("c")
    sid = lax.axis_index("s")

    @pl.when((cid == 0) & (sid == 0))
    def _():
        pltpu.sync_copy(pos_hbm, s_v.at[pl.ds(0, 2 * _N)])
        pltpu.sync_copy(vel_hbm, s_v.at[pl.ds(2 * _N, 2 * _N)])
        pltpu.sync_copy(rec_hbm, rec_v)

        lane = lax.iota(jnp.int32, _L)
        selpat = (((lane >= 4) & (lane < 8)) | (lane == 10)
                  | (lane == 11)).astype(jnp.int32)
        off3 = jnp.where(
            lane < 8, (lane & 1) + (2 * _N) * ((lane >> 1) & 1),
            jnp.where(lane < 12, (2 * _N) + (lane & 1), 0))
        mask8 = lane < 8
        hi8 = (lane >= 8) & (lane < 12)
        vel4 = ((lane & 2) != 0) & mask8
        zero = jnp.zeros((_L,), jnp.float32)

        def chunk(cc, off):
            rows = cc * _L + lane
            vf = plsc.load_gather(rec_v, [rows * _L + 13])
            jf = plsc.load_gather(rec_v, [rows * _L + 12])
            v = vf.astype(jnp.int32)
            m = v > 0
            rank2 = ((plsc.cumsum(v) - 1) + off) * 2
            plsc.store_scatter(civ_v, [rank2], rows, mask=m)
            plsc.store_scatter(civ_v, [rank2 + 1], jf.astype(jnp.int32),
                               mask=m)
            return off + jnp.sum(v)

        nc = lax.fori_loop(0, _N // _L, chunk, 0)

        def one(t):
            t2 = jnp.full((_L,), 2 * t, jnp.int32)
            ivec = plsc.load_gather(civ_v, [t2])
            sel = plsc.load_gather(civ_v, [t2 + selpat])
            rec = plsc.load_gather(rec_v, [ivec * _L + lane])
            idx = 2 * sel + off3
            state = plsc.load_gather(s_v, [idx])
            tt = rec * state
            vn = jnp.sum(jnp.where(hi8, tt, zero))
            vnb = jnp.full((_L,), vn)
            delta = jnp.where(vel4, jnp.where(vnb < 0.0, vnb * rec, zero),
                              rec)
            plsc.addupdate_scatter(s_v, [idx], delta, mask=mask8)

        def body2(h, carry):
            one(2 * h)
            one(2 * h + 1)
            return carry

        lax.fori_loop(0, nc // 2, body2, 0)

        @pl.when(nc % 2 == 1)
        def _tail():
            one(nc - 1)

        pltpu.sync_copy(s_v.at[pl.ds(0, 2 * _N)], pos_out)
        pltpu.sync_copy(s_v.at[pl.ds(2 * _N, 2 * _N)], vel_out)


def _resolve(*args):
    fn = functools.partial(
        pl.kernel,
        out_type=[jax.ShapeDtypeStruct((2 * _N,), jnp.float32),
                  jax.ShapeDtypeStruct((2 * _N,), jnp.float32)],
        mesh=plsc.VectorSubcoreMesh(core_axis_name="c", subcore_axis_name="s"),
        scratch_types=[
            pltpu.VMEM((4 * _N,), jnp.float32),
            pltpu.VMEM((_N * _L,), jnp.float32),
            pltpu.VMEM((2 * _N,), jnp.int32),
        ],
        compiler_params=pltpu.CompilerParams(needs_layout_passes=False),
    )(_resolve_body)
    return fn(*args)


def kernel(positions, velocities, radii, masses):
    posT = positions.T
    rec = _detect(posT, positions, radii, masses)
    pos_o, vel_o = _resolve(
        positions.reshape(2 * _N), velocities.reshape(2 * _N),
        rec.reshape(_N * _L))
    return jnp.concatenate(
        [pos_o.reshape(_N, 2), vel_o.reshape(_N, 2)], axis=-1)


# speedup vs baseline: 1.0190x; 1.0190x over previous
import functools

import jax
import jax.numpy as jnp
import numpy as np
from jax import lax
from jax.experimental import pallas as pl
from jax.experimental.pallas import tpu as pltpu
from jax.experimental.pallas import tpu_sc as plsc

_N = 1024
_B = 512
_L = 16

_OMU = float(np.float32(1.0) - np.array(1064475214, np.uint32).view(np.float32))


def _detect_body(posT, pos, rr, mr, rmc, rec_ref, lt_ref):
    g = pl.program_id(0)
    i0 = g * _B
    jj = lax.broadcasted_iota(jnp.int32, (_B, _N), 1)
    ii = i0 + lax.broadcasted_iota(jnp.int32, (_B, _N), 0)

    @pl.when(g == 0)
    def _():
        rows = lax.broadcasted_iota(jnp.int32, (_N, _N), 0)
        cols = lax.broadcasted_iota(jnp.int32, (_N, _N), 1)
        lt_ref[...] = (rows <= cols).astype(jnp.bfloat16)

    px_row = posT[0:1, :]
    py_row = posT[1:2, :]
    px_col = pos[:, 0:1]
    py_col = pos[:, 1:2]

    dx = px_row - px_col
    dy = py_row - py_col
    dd = (dx * dx + dy * dy) + 1e-12
    dist = jnp.sqrt(dd)
    rc = rmc[:, 0:1]
    mc = rmc[:, 1:2]
    pen = (rc + rr[...]) - dist
    mask = (pen > 0.0) & (jj < ii)

    mif = mask.astype(jnp.float32)
    k = lax.dot_general(
        mask.astype(jnp.bfloat16), lt_ref[...],
        (((1,), (0,)), ((), ())), preferred_element_type=jnp.float32)

    cnt = k[:, _N - 1:_N]
    cnt_f = jnp.maximum(cnt, 1.0)
    q = 1.0 / cnt_f
    c = k * q
    r = (cnt_f * q) * _OMU
    onehot = (c >= r) & (((k - mif) * q) < r)
    ohf = onehot.astype(jnp.float32)

    jsel = jnp.sum(onehot.astype(jnp.int32) * jj, axis=1, keepdims=True)
    dxs = jnp.sum(ohf * dx, axis=1, keepdims=True)
    dys = jnp.sum(ohf * dy, axis=1, keepdims=True)
    dists = jnp.sum(ohf * dist, axis=1, keepdims=True)
    pens = jnp.sum(ohf * pen, axis=1, keepdims=True)
    imr = 1.0 / mr[...]
    imj = jnp.sum(ohf * imr, axis=1, keepdims=True)
    imi = 1.0 / mc

    dists = jnp.where(dists > 0.0, dists, 1.0)
    pvx = dxs / dists * pens
    pvy = dys / dists * pens
    nden = jnp.sqrt(pvx * pvx + pvy * pvy) + 1e-12
    nxv = pvx / nden
    nyv = pvy / nden
    s = imi + imj
    avi = 1.5 * imi / s
    avj = 1.5 * imj / s
    corrx = (0.8 * pvx) / s
    corry = (0.8 * pvy) / s

    z = jnp.zeros((_B, 1), jnp.float32)
    rec = jnp.concatenate(
        [-(corrx * imi), -(corry * imi), avi * nxv, avi * nyv,
         corrx * imj, corry * imj, -(avj * nxv), -(avj * nyv),
         -nxv, -nyv, nxv, nyv,
         jsel.astype(jnp.float32), (cnt > 0.0).astype(jnp.float32),
         z, z], axis=1)
    rec_ref[...] = rec


def _detect(posT, pos, radii, masses):
    row2 = pl.BlockSpec((2, _N), lambda g: (0, 0))
    row1 = pl.BlockSpec((1, _N), lambda g: (0, 0))
    col2 = pl.BlockSpec((_B, 2), lambda g: (g, 0))
    col1 = pl.BlockSpec((_B, 1), lambda g: (g, 0))
    outw = pl.BlockSpec((_B, _L), lambda g: (g, 0))
    fw = jax.ShapeDtypeStruct((_N, _L), jnp.float32)
    rm = jnp.concatenate([radii[:, None], masses[:, None]], axis=1)
    return pl.pallas_call(
        _detect_body,
        grid=(_N // _B,),
        in_specs=[row2, col2, row1, row1, col2],
        out_specs=[outw],
        out_shape=[fw],
        scratch_shapes=[pltpu.VMEM((_N, _N), jnp.bfloat16)],
    )(posT, pos, radii.reshape(1, _N), masses.reshape(1, _N), rm)[0]


def _resolve_body(pos_hbm, vel_hbm, rec_hbm,
                  pos_out, vel_out,
                  s_v, rec_v, civ_v):
    cid = lax.axis_index("c")
    sid = lax.axis_index("s")

    @pl.when((cid == 0) & (sid == 0))
    def _():
        pltpu.sync_copy(pos_hbm, s_v.at[pl.ds(0, 2 * _N)])
        pltpu.sync_copy(vel_hbm, s_v.at[pl.ds(2 * _N, 2 * _N)])
        pltpu.sync_copy(rec_hbm, rec_v)

        lane = lax.iota(jnp.int32, _L)
        selpat = (((lane >= 4) & (lane < 8)) | (lane == 10)
                  | (lane == 11)).astype(jnp.int32)
        off3 = jnp.where(
            lane < 8, (lane & 1) + (2 * _N) * ((lane >> 1) & 1),
            jnp.where(lane < 12, (2 * _N) + (lane & 1), 0))
        mask8 = lane < 8
        hi8 = (lane >= 8) & (lane < 12)
        vel4 = ((lane & 2) != 0) & mask8
        zero = jnp.zeros((_L,), jnp.float32)

        def chunk(cc, off):
            rows = cc * _L + lane
            vf = plsc.load_gather(rec_v, [rows * _L + 13])
            jf = plsc.load_gather(rec_v, [rows * _L + 12])
            v = vf.astype(jnp.int32)
            m = v > 0
            rank2 = ((plsc.cumsum(v) - 1) + off) * 2
            plsc.store_scatter(civ_v, [rank2], rows, mask=m)
            plsc.store_scatter(civ_v, [rank2 + 1], jf.astype(jnp.int32),
                               mask=m)
            return off + jnp.sum(v)

        nc = lax.fori_loop(0, _N // _L, chunk, 0)

        def one(t):
            t2 = jnp.full((_L,), 2 * t, jnp.int32)
            ivec = plsc.load_gather(civ_v, [t2])
            sel = plsc.load_gather(civ_v, [t2 + selpat])
            rec = plsc.load_gather(rec_v, [ivec * _L + lane])
            idx = 2 * sel + off3
            state = plsc.load_gather(s_v, [idx])
            tt = rec * state
            vn = jnp.sum(jnp.where(hi8, tt, zero))
            vnb = jnp.full((_L,), vn)
            delta = jnp.where(vel4, jnp.where(vnb < 0.0, vnb * rec, zero),
                              rec)
            plsc.addupdate_scatter(s_v, [idx], delta, mask=mask8)

        def body(t, carry):
            one(t)
            return carry

        lax.fori_loop(0, nc, body, 0)

        pltpu.sync_copy(s_v.at[pl.ds(0, 2 * _N)], pos_out)
        pltpu.sync_copy(s_v.at[pl.ds(2 * _N, 2 * _N)], vel_out)


def _resolve(*args):
    fn = functools.partial(
        pl.kernel,
        out_type=[jax.ShapeDtypeStruct((2 * _N,), jnp.float32),
                  jax.ShapeDtypeStruct((2 * _N,), jnp.float32)],
        mesh=plsc.VectorSubcoreMesh(core_axis_name="c", subcore_axis_name="s"),
        scratch_types=[
            pltpu.VMEM((4 * _N,), jnp.float32),
            pltpu.VMEM((_N * _L,), jnp.float32),
            pltpu.VMEM((2 * _N,), jnp.int32),
        ],
        compiler_params=pltpu.CompilerParams(needs_layout_passes=False),
    )(_resolve_body)
    return fn(*args)


def kernel(positions, velocities, radii, masses):
    posT = positions.T
    rec = _detect(posT, positions, radii, masses)
    pos_o, vel_o = _resolve(
        positions.reshape(2 * _N), velocities.reshape(2 * _N),
        rec.reshape(_N * _L))
    return jnp.concatenate(
        [pos_o.reshape(_N, 2), vel_o.reshape(_N, 2)], axis=-1)


# speedup vs baseline: 1.0273x; 1.0081x over previous
import functools

import jax
import jax.numpy as jnp
import numpy as np
from jax import lax
from jax.experimental import pallas as pl
from jax.experimental.pallas import tpu as pltpu
from jax.experimental.pallas import tpu_sc as plsc

_N = 1024
_B = 512
_L = 16

_OMU = float(np.float32(1.0) - np.array(1064475214, np.uint32).view(np.float32))


def _detect_body(posT, pos, rr, mr, rmc, rec_ref, lt_ref):
    g = pl.program_id(0)
    i0 = g * _B
    jj = lax.broadcasted_iota(jnp.int32, (_B, _N), 1)
    ii = i0 + lax.broadcasted_iota(jnp.int32, (_B, _N), 0)

    @pl.when(g == 0)
    def _():
        rows = lax.broadcasted_iota(jnp.int32, (_N, _N), 0)
        cols = lax.broadcasted_iota(jnp.int32, (_N, _N), 1)
        lt_ref[...] = (rows <= cols).astype(jnp.bfloat16)

    px_row = posT[0:1, :]
    py_row = posT[1:2, :]
    px_col = pos[:, 0:1]
    py_col = pos[:, 1:2]

    dx = px_row - px_col
    dy = py_row - py_col
    dd = (dx * dx + dy * dy) + 1e-12
    dist = jnp.sqrt(dd)
    rc = rmc[:, 0:1]
    mc = rmc[:, 1:2]
    pen = (rc + rr[...]) - dist
    mask = (pen > 0.0) & (jj < ii)

    k = lax.dot_general(
        mask.astype(jnp.bfloat16), lt_ref[...],
        (((1,), (0,)), ((), ())), preferred_element_type=jnp.float32)

    cnt = k[:, _N - 1:_N]
    cnt_f = jnp.maximum(cnt, 1.0)
    q = 1.0 / cnt_f
    r = (cnt_f * q) * _OMU
    m0 = jnp.floor(r / q)
    mstar = m0 + 2.0
    for cand in (m0 + 1.0, m0, jnp.maximum(m0 - 1.0, 1.0)):
        mstar = jnp.where((cand * q) >= r, cand, mstar)
    mstar = jnp.minimum(jnp.maximum(mstar, 1.0), cnt_f)
    onehot = (k == mstar) & mask
    ohf = onehot.astype(jnp.float32)

    jsel = jnp.sum(onehot.astype(jnp.int32) * jj, axis=1, keepdims=True)
    dxs = jnp.sum(ohf * dx, axis=1, keepdims=True)
    dys = jnp.sum(ohf * dy, axis=1, keepdims=True)
    dists = jnp.sum(ohf * dist, axis=1, keepdims=True)
    pens = jnp.sum(ohf * pen, axis=1, keepdims=True)
    imr = 1.0 / mr[...]
    imj = jnp.sum(ohf * imr, axis=1, keepdims=True)
    imi = 1.0 / mc

    dists = jnp.where(dists > 0.0, dists, 1.0)
    pvx = dxs / dists * pens
    pvy = dys / dists * pens
    nden = jnp.sqrt(pvx * pvx + pvy * pvy) + 1e-12
    nxv = pvx / nden
    nyv = pvy / nden
    s = imi + imj
    avi = 1.5 * imi / s
    avj = 1.5 * imj / s
    corrx = (0.8 * pvx) / s
    corry = (0.8 * pvy) / s

    z = jnp.zeros((_B, 1), jnp.float32)
    rec = jnp.concatenate(
        [-(corrx * imi), -(corry * imi), avi * nxv, avi * nyv,
         corrx * imj, corry * imj, -(avj * nxv), -(avj * nyv),
         -nxv, -nyv, nxv, nyv,
         jsel.astype(jnp.float32), (cnt > 0.0).astype(jnp.float32),
         z, z], axis=1)
    rec_ref[...] = rec


def _detect(posT, pos, radii, masses):
    row2 = pl.BlockSpec((2, _N), lambda g: (0, 0))
    row1 = pl.BlockSpec((1, _N), lambda g: (0, 0))
    col2 = pl.BlockSpec((_B, 2), lambda g: (g, 0))
    col1 = pl.BlockSpec((_B, 1), lambda g: (g, 0))
    outw = pl.BlockSpec((_B, _L), lambda g: (g, 0))
    fw = jax.ShapeDtypeStruct((_N, _L), jnp.float32)
    rm = jnp.concatenate([radii[:, None], masses[:, None]], axis=1)
    return pl.pallas_call(
        _detect_body,
        grid=(_N // _B,),
        in_specs=[row2, col2, row1, row1, col2],
        out_specs=[outw],
        out_shape=[fw],
        scratch_shapes=[pltpu.VMEM((_N, _N), jnp.bfloat16)],
    )(posT, pos, radii.reshape(1, _N), masses.reshape(1, _N), rm)[0]


def _resolve_body(pos_hbm, vel_hbm, rec_hbm,
                  pos_out, vel_out,
                  s_v, rec_v, civ_v):
    cid = lax.axis_index("c")
    sid = lax.axis_index("s")

    @pl.when((cid == 0) & (sid == 0))
    def _():
        pltpu.sync_copy(pos_hbm, s_v.at[pl.ds(0, 2 * _N)])
        pltpu.sync_copy(vel_hbm, s_v.at[pl.ds(2 * _N, 2 * _N)])
        pltpu.sync_copy(rec_hbm, rec_v)

        lane = lax.iota(jnp.int32, _L)
        selpat = (((lane >= 4) & (lane < 8)) | (lane == 10)
                  | (lane == 11)).astype(jnp.int32)
        off3 = jnp.where(
            lane < 8, (lane & 1) + (2 * _N) * ((lane >> 1) & 1),
            jnp.where(lane < 12, (2 * _N) + (lane & 1), 0))
        mask8 = lane < 8
        hi8 = (lane >= 8) & (lane < 12)
        vel4 = ((lane & 2) != 0) & mask8
        zero = jnp.zeros((_L,), jnp.float32)

        def chunk(cc, off):
            rows = cc * _L + lane
            vf = plsc.load_gather(rec_v, [rows * _L + 13])
            jf = plsc.load_gather(rec_v, [rows * _L + 12])
            v = vf.astype(jnp.int32)
            m = v > 0
            rank2 = ((plsc.cumsum(v) - 1) + off) * 2
            plsc.store_scatter(civ_v, [rank2], rows, mask=m)
            plsc.store_scatter(civ_v, [rank2 + 1], jf.astype(jnp.int32),
                               mask=m)
            return off + jnp.sum(v)

        nc = lax.fori_loop(0, _N // _L, chunk, 0)

        def one(t):
            t2 = jnp.full((_L,), 2 * t, jnp.int32)
            ivec = plsc.load_gather(civ_v, [t2])
            sel = plsc.load_gather(civ_v, [t2 + selpat])
            rec = plsc.load_gather(rec_v, [ivec * _L + lane])
            idx = 2 * sel + off3
            state = plsc.load_gather(s_v, [idx])
            tt = rec * state
            vn = jnp.sum(jnp.where(hi8, tt, zero))
            vnb = jnp.full((_L,), vn)
            delta = jnp.where(vel4, jnp.where(vnb < 0.0, vnb * rec, zero),
                              rec)
            plsc.addupdate_scatter(s_v, [idx], delta, mask=mask8)

        def body(t, carry):
            one(t)
            return carry

        lax.fori_loop(0, nc, body, 0)

        pltpu.sync_copy(s_v.at[pl.ds(0, 2 * _N)], pos_out)
        pltpu.sync_copy(s_v.at[pl.ds(2 * _N, 2 * _N)], vel_out)


def _resolve(*args):
    fn = functools.partial(
        pl.kernel,
        out_type=[jax.ShapeDtypeStruct((2 * _N,), jnp.float32),
                  jax.ShapeDtypeStruct((2 * _N,), jnp.float32)],
        mesh=plsc.VectorSubcoreMesh(core_axis_name="c", subcore_axis_name="s"),
        scratch_types=[
            pltpu.VMEM((4 * _N,), jnp.float32),
            pltpu.VMEM((_N * _L,), jnp.float32),
            pltpu.VMEM((2 * _N,), jnp.int32),
        ],
        compiler_params=pltpu.CompilerParams(needs_layout_passes=False),
    )(_resolve_body)
    return fn(*args)


def kernel(positions, velocities, radii, masses):
    posT = positions.T
    rec = _detect(posT, positions, radii, masses)
    pos_o, vel_o = _resolve(
        positions.reshape(2 * _N), velocities.reshape(2 * _N),
        rec.reshape(_N * _L))
    return jnp.concatenate(
        [pos_o.reshape(_N, 2), vel_o.reshape(_N, 2)], axis=-1)


# speedup vs baseline: 1.0478x; 1.0200x over previous
import functools

import jax
import jax.numpy as jnp
import numpy as np
from jax import lax
from jax.experimental import pallas as pl
from jax.experimental.pallas import tpu as pltpu
from jax.experimental.pallas import tpu_sc as plsc

_N = 1024
_B = 512
_L = 16

_OMU = float(np.float32(1.0) - np.array(1064475214, np.uint32).view(np.float32))


def _detect_body(posT, pos, rr, mr, rmc, rec_ref, lt_ref):
    g = pl.program_id(0)
    i0 = g * _B
    jj = lax.broadcasted_iota(jnp.int32, (_B, _N), 1)
    ii = i0 + lax.broadcasted_iota(jnp.int32, (_B, _N), 0)

    @pl.when(g == 0)
    def _():
        rows = lax.broadcasted_iota(jnp.int32, (_N, _N), 0)
        cols = lax.broadcasted_iota(jnp.int32, (_N, _N), 1)
        lt_ref[...] = (rows <= cols).astype(jnp.bfloat16)

    px_row = posT[0:1, :]
    py_row = posT[1:2, :]
    px_col = pos[:, 0:1]
    py_col = pos[:, 1:2]

    dx = px_row - px_col
    dy = py_row - py_col
    dd = (dx * dx + dy * dy) + 1e-12
    dist = jnp.sqrt(dd)
    rc = rmc[:, 0:1]
    mc = rmc[:, 1:2]
    pen = (rc + rr[...]) - dist
    mask = (pen > 0.0) & (jj < ii)

    k = lax.dot_general(
        mask.astype(jnp.bfloat16), lt_ref[...],
        (((1,), (0,)), ((), ())), preferred_element_type=jnp.float32)

    cnt = k[:, _N - 1:_N]
    cnt_f = jnp.maximum(cnt, 1.0)
    q = 1.0 / cnt_f
    r = (cnt_f * q) * _OMU
    m0 = jnp.floor(r / q)
    mstar = m0 + 2.0
    for cand in (m0 + 1.0, m0, jnp.maximum(m0 - 1.0, 1.0)):
        mstar = jnp.where((cand * q) >= r, cand, mstar)
    mstar = jnp.minimum(jnp.maximum(mstar, 1.0), cnt_f)
    onehot = (k == mstar) & mask
    ohf = onehot.astype(jnp.float32)

    jsel = jnp.sum(onehot.astype(jnp.int32) * jj, axis=1, keepdims=True)
    dxs = jnp.sum(ohf * dx, axis=1, keepdims=True)
    dys = jnp.sum(ohf * dy, axis=1, keepdims=True)
    dists = jnp.sum(ohf * dist, axis=1, keepdims=True)
    pens = jnp.sum(ohf * pen, axis=1, keepdims=True)
    imr = 1.0 / mr[...]
    imj = jnp.sum(ohf * imr, axis=1, keepdims=True)
    imi = 1.0 / mc

    dists = jnp.where(dists > 0.0, dists, 1.0)
    pvx = dxs / dists * pens
    pvy = dys / dists * pens
    nden = jnp.sqrt(pvx * pvx + pvy * pvy) + 1e-12
    nxv = pvx / nden
    nyv = pvy / nden
    s = imi + imj
    avi = 1.5 * imi / s
    avj = 1.5 * imj / s
    corrx = (0.8 * pvx) / s
    corry = (0.8 * pvy) / s

    z = jnp.zeros((_B, 1), jnp.float32)
    rec = jnp.concatenate(
        [-(corrx * imi), -(corry * imi), avi * nxv, avi * nyv,
         corrx * imj, corry * imj, -(avj * nxv), -(avj * nyv),
         -nxv, -nyv, nxv, nyv,
         jsel.astype(jnp.float32), (cnt > 0.0).astype(jnp.float32),
         z, z], axis=1)
    rec_ref[...] = rec


def _detect(posT, pos, radii, masses):
    row2 = pl.BlockSpec((2, _N), lambda g: (0, 0))
    row1 = pl.BlockSpec((1, _N), lambda g: (0, 0))
    col2 = pl.BlockSpec((_B, 2), lambda g: (g, 0))
    col1 = pl.BlockSpec((_B, 1), lambda g: (g, 0))
    outw = pl.BlockSpec((_B, _L), lambda g: (g, 0))
    fw = jax.ShapeDtypeStruct((_N, _L), jnp.float32)
    rm = jnp.concatenate([radii[:, None], masses[:, None]], axis=1)
    return pl.pallas_call(
        _detect_body,
        grid=(_N // _B,),
        in_specs=[row2, col2, row1, row1, col2],
        out_specs=[outw],
        out_shape=[fw],
        scratch_shapes=[pltpu.VMEM((_N, _N), jnp.bfloat16)],
    )(posT, pos, radii.reshape(1, _N), masses.reshape(1, _N), rm)[0]


def _resolve_body(pos_hbm, vel_hbm, rec_hbm,
                  pos_out, vel_out,
                  s_v, rec_v, civ_v, occ_v, isoi_v, isoj_v):
    cid = lax.axis_index("c")
    sid = lax.axis_index("s")

    @pl.when((cid == 0) & (sid == 0))
    def _():
        pltpu.sync_copy(pos_hbm, s_v.at[pl.ds(0, 2 * _N)])
        pltpu.sync_copy(vel_hbm, s_v.at[pl.ds(2 * _N, 2 * _N)])
        pltpu.sync_copy(rec_hbm, rec_v)

        lane = lax.iota(jnp.int32, _L)
        selpat = (((lane >= 4) & (lane < 8)) | (lane == 10)
                  | (lane == 11)).astype(jnp.int32)
        off3 = jnp.where(
            lane < 8, (lane & 1) + (2 * _N) * ((lane >> 1) & 1),
            jnp.where(lane < 12, (2 * _N) + (lane & 1), 0))
        mask8 = lane < 8
        hi8 = (lane >= 8) & (lane < 12)
        vel4 = ((lane & 2) != 0) & mask8
        zero = jnp.zeros((_L,), jnp.float32)
        izero = jnp.zeros((_L,), jnp.int32)
        ones = jnp.full((_L,), 1, jnp.int32)

        def occ_pass(cc, carry):
            base = cc * _L
            rows = base + lane
            vf = plsc.load_gather(rec_v, [rows * _L + 13])
            jf = plsc.load_gather(rec_v, [rows * _L + 12])
            v = vf.astype(jnp.int32)
            occ_v[pl.ds(base, _L)] = v
            plsc.addupdate_scatter(occ_v, [jf.astype(jnp.int32)], ones,
                                   mask=v > 0)
            return carry

        lax.fori_loop(0, _N // _L, occ_pass, 0)

        def classify(cc, carry):
            ni, nch = carry
            base = cc * _L
            rows = base + lane
            vf = plsc.load_gather(rec_v, [rows * _L + 13])
            jf = plsc.load_gather(rec_v, [rows * _L + 12])
            jv = jf.astype(jnp.int32)
            m = vf > 0.0
            occ_i = occ_v[pl.ds(base, _L)]
            occ_j = plsc.load_gather(occ_v, [jv])
            iso = m & (occ_i == 1) & (occ_j == 1)
            ch = m & ((occ_i != 1) | (occ_j != 1))
            isoi = iso.astype(jnp.int32)
            chi = ch.astype(jnp.int32)
            riso = (plsc.cumsum(isoi) - 1) + ni
            rch2 = ((plsc.cumsum(chi) - 1) + nch) * 2
            plsc.store_scatter(isoi_v, [riso], rows, mask=iso)
            plsc.store_scatter(isoj_v, [riso], jv, mask=iso)
            plsc.store_scatter(civ_v, [rch2], rows, mask=ch)
            plsc.store_scatter(civ_v, [rch2 + 1], jv, mask=ch)
            return ni + jnp.sum(isoi), nch + jnp.sum(chi)

        ni, nch = lax.fori_loop(0, _N // _L, classify, (0, 0))

        plsc.store_scatter(isoi_v, [ni + lane], izero)
        plsc.store_scatter(isoj_v, [ni + lane], izero)

        def iso_batch(b, carry):
            base = b * _L
            iv = isoi_v[pl.ds(base, _L)]
            jv = isoj_v[pl.ds(base, _L)]
            ri = iv * _L
            rj2 = 2 * jv
            ri2 = 2 * iv
            wnx = plsc.load_gather(rec_v, [ri + 8])
            wny = plsc.load_gather(rec_v, [ri + 9])
            nx = plsc.load_gather(rec_v, [ri + 10])
            ny = plsc.load_gather(rec_v, [ri + 11])
            anx = plsc.load_gather(rec_v, [ri + 2])
            any_ = plsc.load_gather(rec_v, [ri + 3])
            bnx = plsc.load_gather(rec_v, [ri + 6])
            bny = plsc.load_gather(rec_v, [ri + 7])
            dpix = plsc.load_gather(rec_v, [ri])
            dpiy = plsc.load_gather(rec_v, [ri + 1])
            dpjx = plsc.load_gather(rec_v, [ri + 4])
            dpjy = plsc.load_gather(rec_v, [ri + 5])
            vxi = plsc.load_gather(s_v, [2 * _N + ri2])
            vyi = plsc.load_gather(s_v, [2 * _N + ri2 + 1])
            vxj = plsc.load_gather(s_v, [2 * _N + rj2])
            vyj = plsc.load_gather(s_v, [2 * _N + rj2 + 1])
            vn = ((wnx * vxi + wny * vyi) + nx * vxj) + ny * vyj
            neg = vn < 0.0
            plsc.addupdate_scatter(s_v, [ri2], dpix)
            plsc.addupdate_scatter(s_v, [ri2 + 1], dpiy)
            plsc.addupdate_scatter(s_v, [rj2], dpjx)
            plsc.addupdate_scatter(s_v, [rj2 + 1], dpjy)
            plsc.addupdate_scatter(
                s_v, [2 * _N + ri2], jnp.where(neg, vn * anx, zero))
            plsc.addupdate_scatter(
                s_v, [2 * _N + ri2 + 1], jnp.where(neg, vn * any_, zero))
            plsc.addupdate_scatter(
                s_v, [2 * _N + rj2], jnp.where(neg, vn * bnx, zero))
            plsc.addupdate_scatter(
                s_v, [2 * _N + rj2 + 1], jnp.where(neg, vn * bny, zero))
            return carry

        lax.fori_loop(0, (ni + _L - 1) // _L, iso_batch, 0)

        def one(t):
            t2 = jnp.full((_L,), 2 * t, jnp.int32)
            ivec = plsc.load_gather(civ_v, [t2])
            sel = plsc.load_gather(civ_v, [t2 + selpat])
            rec = plsc.load_gather(rec_v, [ivec * _L + lane])
            idx = 2 * sel + off3
            state = plsc.load_gather(s_v, [idx])
            tt = rec * state
            vn = jnp.sum(jnp.where(hi8, tt, zero))
            vnb = jnp.full((_L,), vn)
            delta = jnp.where(vel4, jnp.where(vnb < 0.0, vnb * rec, zero),
                              rec)
            plsc.addupdate_scatter(s_v, [idx], delta, mask=mask8)

        def body(t, carry):
            one(t)
            return carry

        lax.fori_loop(0, nch, body, 0)

        pltpu.sync_copy(s_v.at[pl.ds(0, 2 * _N)], pos_out)
        pltpu.sync_copy(s_v.at[pl.ds(2 * _N, 2 * _N)], vel_out)


def _resolve(*args):
    fn = functools.partial(
        pl.kernel,
        out_type=[jax.ShapeDtypeStruct((2 * _N,), jnp.float32),
                  jax.ShapeDtypeStruct((2 * _N,), jnp.float32)],
        mesh=plsc.VectorSubcoreMesh(core_axis_name="c", subcore_axis_name="s"),
        scratch_types=[
            pltpu.VMEM((4 * _N,), jnp.float32),
            pltpu.VMEM((_N * _L,), jnp.float32),
            pltpu.VMEM((2 * _N,), jnp.int32),
            pltpu.VMEM((_N,), jnp.int32),
            pltpu.VMEM((_N + _L,), jnp.int32),
            pltpu.VMEM((_N + _L,), jnp.int32),
        ],
        compiler_params=pltpu.CompilerParams(needs_layout_passes=False),
    )(_resolve_body)
    return fn(*args)


def kernel(positions, velocities, radii, masses):
    posT = positions.T
    rec = _detect(posT, positions, radii, masses)
    pos_o, vel_o = _resolve(
        positions.reshape(2 * _N), velocities.reshape(2 * _N),
        rec.reshape(_N * _L))
    return jnp.concatenate(
        [pos_o.reshape(_N, 2), vel_o.reshape(_N, 2)], axis=-1)
